# Initial kernel scaffold; baseline (speedup 1.0000x reference)
import jax
import jax.numpy as jnp
from jax.experimental import pallas as pl


def kernel(x_pfc, x_vtx, params, batch_pfc, batch_vtx):
    raise NotImplementedError("write your pallas kernel here")


# speedup vs baseline: 5.1946x; 5.1946x over previous
import functools

import jax
import jax.numpy as jnp
from jax import lax
from jax.experimental import pallas as pl
from jax.experimental.pallas import tpu as pltpu
from jax.experimental.pallas import tpu_sc as plsc

N_PFC = 4096
N_VTX = 512
NB = 8
HID = 160
BIG = 1e10
RB = 256
CK = 128
NCHUNK = N_PFC // CK


def _silu(x):
    return x / (1.0 + jnp.exp(-x))


def _full(shape):
    return pl.BlockSpec(shape, lambda i: tuple(0 for _ in shape))


def _dot(a, b):
    return jax.lax.dot_general(a, b, (((1,), (0,)), ((), ())),
                               preferred_element_type=jnp.float32)


def _prep_body(xp_ref, c1w, c1b, c2w, c2b, n1w, n1b, n2w, n2b,
               daw, dab, dbw, x0_ref, a1_ref, g1_ref, chf_ref):
    xp = xp_ref[...]
    chf = (xp[:, 10:11] != 0.0).astype(jnp.float32)
    ch = _silu(_dot(xp, c1w[...]) + c1b[...])
    ch = _dot(ch, c2w[...]) + c2b[...]
    ne = _silu(_dot(xp, n1w[...]) + n1b[...])
    ne = _dot(ne, n2w[...]) + n2b[...]
    x0 = ch * chf + ne * (1.0 - chf)
    x0_ref[...] = x0
    a1_ref[...] = _dot(x0, daw[...]) + dab[...]
    g1_ref[...] = _dot(x0, dbw[...])
    chf_ref[...] = chf


def _run_prep(x_pfc, p, n1w_pad, d1diff, d1b_w):
    row = lambda v: v.reshape(1, -1)
    out_shape = [
        jax.ShapeDtypeStruct((N_PFC, HID), jnp.float32),
        jax.ShapeDtypeStruct((N_PFC, HID), jnp.float32),
        jax.ShapeDtypeStruct((N_PFC, HID), jnp.float32),
        jax.ShapeDtypeStruct((N_PFC, 1), jnp.float32),
    ]
    blk = lambda r, c: pl.BlockSpec((r, c), lambda i: (i, 0))
    return pl.pallas_call(
        _prep_body,
        grid=(8,),
        in_specs=[blk(512, 12),
                  _full((12, 80)), _full((1, 80)), _full((80, 160)), _full((1, 160)),
                  _full((12, 80)), _full((1, 80)), _full((80, 160)), _full((1, 160)),
                  _full((160, 160)), _full((1, 160)), _full((160, 160))],
        out_specs=[blk(512, HID), blk(512, HID), blk(512, HID), blk(512, 1)],
        out_shape=out_shape,
    )(x_pfc, p['c_1_w'], row(p['c_1_b']), p['c_2_w'], row(p['c_2_b']),
      n1w_pad, row(p['n_1_b']), p['n_2_w'], row(p['n_2_b']),
      d1diff, row(p['d1_1_b']), d1b_w)


def _knn_body(K, self_mask, xr_ref, xT_ref, bcol_ref, brow_ref, cmask_ref,
              idx_ref, ds_ref):
    i = pl.program_id(0)
    xb = xr_ref[...]
    rn = jnp.sum(xb * xb, axis=1, keepdims=True)
    brow = brow_ref[...]
    b_lo = jnp.min(brow)
    b_hi = jnp.max(brow)
    bcol = bcol_ref[...]
    first = jnp.sum((bcol < b_lo).astype(jnp.int32))
    last = jnp.sum((bcol <= b_hi).astype(jnp.int32))
    c0 = first // CK
    c1 = (last + CK - 1) // CK
    row_g = i * RB + lax.broadcasted_iota(jnp.int32, (RB, 1), 0)

    def fill(c, _):
        xc = xT_ref[:, pl.ds(c * CK, CK)]
        cn = jnp.sum(xc * xc, axis=0, keepdims=True)
        d = rn + cn - 2.0 * _dot(xb, xc)
        gcol = c * CK + lax.broadcasted_iota(jnp.int32, (1, CK), 1)
        bc = bcol_ref[:, pl.ds(c * CK, CK)]
        msk = (bc != brow) | (cmask_ref[:, pl.ds(c * CK, CK)] == 0.0)
        if self_mask:
            msk = msk | (gcol == row_g)
        ds_ref[:, pl.ds(c * CK, CK)] = jnp.where(msk, BIG, d)
        return 0

    lax.fori_loop(c0, c1, fill, 0)

    INF = float(jnp.inf)
    BIGI = 2 ** 30
    prev_v = jnp.full((RB, 1), -INF, jnp.float32)
    prev_i = jnp.full((RB, 1), -1, jnp.int32)
    for k in range(K):
        def sel(c, carry):
            m, mi = carry
            d = ds_ref[:, pl.ds(c * CK, CK)]
            gcol = c * CK + lax.broadcasted_iota(jnp.int32, (RB, CK), 1)
            elig = (d > prev_v) | ((d == prev_v) & (gcol > prev_i))
            dm = jnp.where(elig, d, INF)
            cm = jnp.min(dm, axis=1, keepdims=True)
            cmi = jnp.min(jnp.where(dm == cm, gcol, BIGI), axis=1,
                          keepdims=True)
            better = (cm < m) | ((cm == m) & (cmi < mi))
            return jnp.where(better, cm, m), jnp.where(better, cmi, mi)

        m, mi = lax.fori_loop(
            c0, c1, sel,
            (jnp.full((RB, 1), INF, jnp.float32),
             jnp.full((RB, 1), BIGI, jnp.int32)))
        idx_ref[:, k:k + 1] = mi
        prev_v, prev_i = m, mi


def _run_knn(x, xT, bcol, brow, cmask, K, self_mask):
    body = functools.partial(_knn_body, K, self_mask)
    blk = lambda r, c: pl.BlockSpec((r, c), lambda i: (i, 0))
    return pl.pallas_call(
        body,
        grid=(N_PFC // RB,),
        in_specs=[blk(RB, HID), _full((HID, N_PFC)), _full((1, N_PFC)),
                  blk(RB, 1), _full((1, N_PFC))],
        out_specs=blk(RB, K),
        out_shape=jax.ShapeDtypeStruct((N_PFC, K), jnp.int32),
        scratch_shapes=[pltpu.VMEM((RB, N_PFC), jnp.float32)],
    )(x, xT, bcol, brow, cmask)


def _edge_conv_sc(a, g, idx, K):
    N, D0 = a.shape
    D = 256
    a = jnp.pad(a, ((0, 0), (0, D - D0)))
    g = jnp.pad(g, ((0, 0), (0, D - D0)))
    NC, NS = 2, 16
    NW = NC * NS
    npw = N // NW
    GN = 128 // K
    ngroups = npw // GN
    nreg = D0 // 16
    idxr = idx.reshape(N * K // 128, 128)
    mesh = plsc.VectorSubcoreMesh(core_axis_name="c", subcore_axis_name="s")

    @functools.partial(
        pl.kernel, mesh=mesh,
        out_type=jax.ShapeDtypeStruct((N, D), jnp.float32),
        scratch_types=[pltpu.VMEM((ngroups, 128), jnp.int32),
                       pltpu.VMEM((npw, D), jnp.float32),
                       pltpu.VMEM((128, D), jnp.float32),
                       pltpu.VMEM((npw, D), jnp.float32),
                       pltpu.SemaphoreType.DMA],
    )
    def edge_k(a_hbm, g_hbm, idx_hbm, out_hbm, idx_v, a_v, rows_v, out_v, sem):
        wid = lax.axis_index("s") * NC + lax.axis_index("c")
        base = wid * npw
        pltpu.sync_copy(idx_hbm.at[pl.ds(wid * ngroups, ngroups)], idx_v)
        pltpu.sync_copy(a_hbm.at[pl.ds(base, npw)], a_v)
        inv = jnp.float32(1.0 / K)

        def group(gi, _):
            pltpu.async_copy(g_hbm.at[idx_v.at[gi]], rows_v, sem).wait()
            for n in range(GN):
                node = gi * GN + n
                a_regs = [a_v[node, pl.ds(c * 16, 16)] for c in range(nreg)]

                def kbody(kk, acc):
                    r = n * K + kk
                    out = []
                    for c in range(nreg):
                        t = a_regs[c] + rows_v[r, pl.ds(c * 16, 16)]
                        out.append(acc[c] + t / (1.0 + jnp.exp(-t)))
                    return tuple(out)

                acc = lax.fori_loop(
                    0, K, kbody,
                    tuple(jnp.zeros((16,), jnp.float32) for _ in range(nreg)))
                for c in range(nreg):
                    out_v[node, pl.ds(c * 16, 16)] = acc[c] * inv
            return 0

        lax.fori_loop(0, ngroups, group, 0)
        pltpu.sync_copy(out_v, out_hbm.at[pl.ds(base, npw)])

    return edge_k(a, g, idxr)[:, :D0]


def _mid_body(s_ref, w12, b12, daw, dab, dbw, x1_ref, a2_ref, g2_ref):
    x1 = _dot(s_ref[...], w12[...]) + b12[...]
    x1_ref[...] = x1
    a2_ref[...] = _dot(x1, daw[...]) + dab[...]
    g2_ref[...] = _dot(x1, dbw[...])


def _run_mid(s1, p, d2diff, d2b_w):
    row = lambda v: v.reshape(1, -1)
    blk = lambda r, c: pl.BlockSpec((r, c), lambda i: (i, 0))
    return pl.pallas_call(
        _mid_body,
        grid=(8,),
        in_specs=[blk(512, HID), _full((160, 160)), _full((1, 160)),
                  _full((160, 160)), _full((1, 160)), _full((160, 160))],
        out_specs=[blk(512, HID)] * 3,
        out_shape=[jax.ShapeDtypeStruct((N_PFC, HID), jnp.float32)] * 3,
    )(s1, p['d1_2_w'], row(p['d1_2_b']), d2diff, row(p['d2_1_b']), d2b_w)


def _vtx_body(xv_ref, bv_ref, w11, b11, w12, b12, w13, b13,
              w21, b21, w22, b22, euc_ref, t_ref, vf_ref):
    xv = xv_ref[...]
    h = _silu(_dot(xv, w11[...]) + b11[...])
    h = _silu(_dot(h, w12[...]) + b12[...])
    euc = _dot(h, w13[...]) + b13[...]
    euc_ref[...] = euc
    vf = _dot(_silu(_dot(euc, w21[...]) + b21[...]), w22[...]) + b22[...]
    vf_ref[...] = vf
    bv = bv_ref[...]
    for b in range(NB):
        fb = jnp.sum((bv < b).astype(jnp.int32))
        fb = jnp.minimum(fb, N_VTX - 1)
        t_ref[b:b + 1, :] = vf_ref[pl.ds(fb, 1), :]


def _run_vtx(x_vtx, bv, p):
    row = lambda v: v.reshape(1, -1)
    return pl.pallas_call(
        _vtx_body,
        grid=(1,),
        in_specs=[_full((N_VTX, 5)), _full((N_VTX, 1)),
                  _full((5, 40)), _full((1, 40)), _full((40, 80)),
                  _full((1, 80)), _full((80, 160)), _full((1, 160)),
                  _full((160, 320)), _full((1, 320)), _full((320, 160)),
                  _full((1, 160))],
        out_specs=[_full((N_VTX, HID)), _full((NB, HID))],
        out_shape=[jax.ShapeDtypeStruct((N_VTX, HID), jnp.float32),
                   jax.ShapeDtypeStruct((NB, HID), jnp.float32)],
        scratch_shapes=[pltpu.VMEM((N_VTX, HID), jnp.float32)],
    )(x_vtx, bv, p['v1_1_w'], row(p['v1_1_b']), p['v1_2_w'], row(p['v1_2_b']),
      p['v1_3_w'], row(p['v1_3_b']), p['v2_1_w'], row(p['v2_1_b']),
      p['v2_2_w'], row(p['v2_2_b']))


def _final_body(s2_ref, bp_ref, w22d, b22d, p1w, p1b, p2w, p2b, t_ref,
                sc_ref):
    feats2 = _dot(s2_ref[...], w22d[...]) + b22d[...]
    h = _silu(_dot(feats2, p1w[...]) + p1b[...])
    pf = _dot(h, p2w[...]) + p2b[...]
    bp = bp_ref[...]
    oh = (bp == lax.broadcasted_iota(jnp.int32, (1, NB), 1)).astype(
        jnp.float32)
    tgt = _dot(oh, t_ref[...])
    s0 = jnp.sum(pf * tgt, axis=1, keepdims=True)
    sc_ref[...] = jnp.concatenate([s0, -s0], axis=1)


def _run_final(s2, bp, p, t):
    row = lambda v: v.reshape(1, -1)
    blk = lambda r, c: pl.BlockSpec((r, c), lambda i: (i, 0))
    return pl.pallas_call(
        _final_body,
        grid=(8,),
        in_specs=[blk(512, HID), blk(512, 1), _full((160, 160)),
                  _full((1, 160)), _full((160, 320)), _full((1, 320)),
                  _full((320, 160)), _full((1, 160)), _full((NB, HID))],
        out_specs=blk(512, 2),
        out_shape=jax.ShapeDtypeStruct((N_PFC, 2), jnp.float32),
    )(s2, bp, p['d2_2_w'], row(p['d2_2_b']), p['p2_1_w'], row(p['p2_1_b']),
      p['p2_2_w'], row(p['p2_2_b']), t)


def kernel(x_pfc, x_vtx, params, batch_pfc, batch_vtx):
    p = params
    f32 = jnp.float32
    n1w_pad = jnp.concatenate([p['n_1_w'], jnp.zeros((1, 80), f32)], axis=0)
    d1diff = p['d1_1_w'][:HID] - p['d1_1_w'][HID:]
    d2diff = p['d2_1_w'][:HID] - p['d2_1_w'][HID:]
    bp_col = batch_pfc.reshape(-1, 1)
    bp_row = batch_pfc.reshape(1, -1)
    bv_col = batch_vtx.reshape(-1, 1)
    ones = jnp.ones((1, N_PFC), f32)

    x0, a1, g1, chf = _run_prep(x_pfc, p, n1w_pad, d1diff, p['d1_1_w'][HID:])
    idx1 = _run_knn(x0, x0.T, bp_row, bp_col, ones, 32, True)
    s1 = _edge_conv_sc(a1, g1, idx1, 32)
    x1, a2, g2 = _run_mid(s1, p, d2diff, p['d2_1_w'][HID:])
    idx2 = _run_knn(x1, x1.T, bp_row, bp_col, chf.reshape(1, -1), 16, False)
    s2 = _edge_conv_sc(a2, g2, idx2, 16)
    euc, t = _run_vtx(x_vtx, bv_col, p)
    scores = _run_final(s2, bp_col, p, t)
    return scores, x1, euc


# speedup vs baseline: 6.0292x; 1.1607x over previous
import functools

import jax
import jax.numpy as jnp
from jax import lax
from jax.experimental import pallas as pl
from jax.experimental.pallas import tpu as pltpu
from jax.experimental.pallas import tpu_sc as plsc

N_PFC = 4096
N_VTX = 512
NB = 8
HID = 160
BIG = 1e10
RB = 256
CK = 128
NCHUNK = N_PFC // CK


def _silu(x):
    return x / (1.0 + jnp.exp(-x))


def _full(shape):
    return pl.BlockSpec(shape, lambda i: tuple(0 for _ in shape))


def _dot(a, b):
    return jax.lax.dot_general(a, b, (((1,), (0,)), ((), ())),
                               preferred_element_type=jnp.float32)


def _prep_body(xp_ref, c1w, c1b, c2w, c2b, n1w, n1b, n2w, n2b,
               daw, dab, dbw, x0_ref, a1_ref, g1_ref, chf_ref):
    xp = xp_ref[...]
    chf = (xp[:, 10:11] != 0.0).astype(jnp.float32)
    ch = _silu(_dot(xp, c1w[...]) + c1b[...])
    ch = _dot(ch, c2w[...]) + c2b[...]
    ne = _silu(_dot(xp, n1w[...]) + n1b[...])
    ne = _dot(ne, n2w[...]) + n2b[...]
    x0 = ch * chf + ne * (1.0 - chf)
    x0_ref[...] = x0
    a1_ref[...] = _dot(x0, daw[...]) + dab[...]
    g1_ref[...] = _dot(x0, dbw[...])
    chf_ref[...] = chf


def _run_prep(x_pfc, p, n1w_pad, d1diff, d1b_w):
    row = lambda v: v.reshape(1, -1)
    out_shape = [
        jax.ShapeDtypeStruct((N_PFC, HID), jnp.float32),
        jax.ShapeDtypeStruct((N_PFC, HID), jnp.float32),
        jax.ShapeDtypeStruct((N_PFC, HID), jnp.float32),
        jax.ShapeDtypeStruct((N_PFC, 1), jnp.float32),
    ]
    blk = lambda r, c: pl.BlockSpec((r, c), lambda i: (i, 0))
    return pl.pallas_call(
        _prep_body,
        grid=(8,),
        in_specs=[blk(512, 12),
                  _full((12, 80)), _full((1, 80)), _full((80, 160)), _full((1, 160)),
                  _full((12, 80)), _full((1, 80)), _full((80, 160)), _full((1, 160)),
                  _full((160, 160)), _full((1, 160)), _full((160, 160))],
        out_specs=[blk(512, HID), blk(512, HID), blk(512, HID), blk(512, 1)],
        out_shape=out_shape,
    )(x_pfc, p['c_1_w'], row(p['c_1_b']), p['c_2_w'], row(p['c_2_b']),
      n1w_pad, row(p['n_1_b']), p['n_2_w'], row(p['n_2_b']),
      d1diff, row(p['d1_1_b']), d1b_w)


def _knn_body(K, self_mask, xr_ref, xT_ref, bcol_ref, brow_ref, cmask_ref,
              idx_ref, ds_ref):
    i = pl.program_id(0)
    xb = xr_ref[...]
    rn = jnp.sum(xb * xb, axis=1, keepdims=True)
    brow = brow_ref[...]
    b_lo = jnp.min(brow)
    b_hi = jnp.max(brow)
    bcol = bcol_ref[...]
    first = jnp.sum((bcol < b_lo).astype(jnp.int32))
    last = jnp.sum((bcol <= b_hi).astype(jnp.int32))
    c0 = first // CK
    c1 = (last + CK - 1) // CK
    row_g = i * RB + lax.broadcasted_iota(jnp.int32, (RB, 1), 0)

    def fill(c, _):
        xc = xT_ref[:, pl.ds(c * CK, CK)]
        cn = jnp.sum(xc * xc, axis=0, keepdims=True)
        d = rn + cn - 2.0 * _dot(xb, xc)
        gcol = c * CK + lax.broadcasted_iota(jnp.int32, (1, CK), 1)
        bc = bcol_ref[:, pl.ds(c * CK, CK)]
        msk = (bc != brow) | (cmask_ref[:, pl.ds(c * CK, CK)] == 0.0)
        if self_mask:
            msk = msk | (gcol == row_g)
        ds_ref[:, pl.ds(c * CK, CK)] = jnp.where(msk, BIG, d)
        return 0

    lax.fori_loop(c0, c1, fill, 0)

    INF = float(jnp.inf)
    BIGI = 2 ** 30
    prev_v = jnp.full((RB, 1), -INF, jnp.float32)
    prev_i = jnp.full((RB, 1), -1, jnp.int32)

    def lex_lt(v, i, w, j):
        return (v < w) | ((v == w) & (i < j))

    npair = (c1 - c0 + 1) // 2

    for k in range(K // 2):
        def chunk2(c, valid):
            d = ds_ref[:, pl.ds(c * CK, CK)]
            gcol = c * CK + lax.broadcasted_iota(jnp.int32, (RB, CK), 1)
            elig = (d > prev_v) | ((d == prev_v) & (gcol > prev_i))
            dm = jnp.where(elig, d, INF)
            cm1 = jnp.min(dm, axis=1, keepdims=True)
            ci1 = jnp.min(jnp.where(dm == cm1, gcol, BIGI), axis=1,
                          keepdims=True)
            dm2 = jnp.where((dm == cm1) & (gcol == ci1), INF, dm)
            cm2 = jnp.min(dm2, axis=1, keepdims=True)
            ci2 = jnp.min(jnp.where(dm2 == cm2, gcol, BIGI), axis=1,
                          keepdims=True)
            if valid is not None:
                cm1 = jnp.where(valid, cm1, INF)
                cm2 = jnp.where(valid, cm2, INF)
                ci1 = jnp.where(valid, ci1, BIGI)
                ci2 = jnp.where(valid, ci2, BIGI)
            return cm1, ci1, cm2, ci2

        def merge2(m1, i1, m2, i2, cm1, ci1, cm2, ci2):
            a_lt = lex_lt(cm1, ci1, m1, i1)
            n1v = jnp.where(a_lt, cm1, m1)
            n1i = jnp.where(a_lt, ci1, i1)
            s1v = jnp.where(a_lt, m1, cm1)
            s1i = jnp.where(a_lt, i1, ci1)
            s2v = jnp.where(a_lt, cm2, m2)
            s2i = jnp.where(a_lt, ci2, i2)
            b_lt = lex_lt(s1v, s1i, s2v, s2i)
            return (n1v, n1i, jnp.where(b_lt, s1v, s2v),
                    jnp.where(b_lt, s1i, s2i))

        def sel(p, carry):
            ca = c0 + 2 * p
            cb = jnp.minimum(ca + 1, NCHUNK - 1)
            ra = chunk2(ca, None)
            rb = chunk2(cb, (ca + 1 < c1))
            m = merge2(*ra, *rb)
            return merge2(*carry, *m)

        m1, i1, m2, i2 = lax.fori_loop(
            0, npair, sel,
            (jnp.full((RB, 1), INF, jnp.float32),
             jnp.full((RB, 1), BIGI, jnp.int32),
             jnp.full((RB, 1), INF, jnp.float32),
             jnp.full((RB, 1), BIGI, jnp.int32)))
        idx_ref[:, 2 * k:2 * k + 1] = i1
        idx_ref[:, 2 * k + 1:2 * k + 2] = i2
        prev_v, prev_i = m2, i2


def _run_knn(x, xT, bcol, brow, cmask, K, self_mask):
    body = functools.partial(_knn_body, K, self_mask)
    blk = lambda r, c: pl.BlockSpec((r, c), lambda i: (i, 0))
    return pl.pallas_call(
        body,
        grid=(N_PFC // RB,),
        in_specs=[blk(RB, HID), _full((HID, N_PFC)), _full((1, N_PFC)),
                  blk(RB, 1), _full((1, N_PFC))],
        out_specs=blk(RB, K),
        out_shape=jax.ShapeDtypeStruct((N_PFC, K), jnp.int32),
        scratch_shapes=[pltpu.VMEM((RB, N_PFC), jnp.float32)],
    )(x, xT, bcol, brow, cmask)


def _edge_conv_sc(a, g, idx, K):
    N, D0 = a.shape
    D = 256
    a = jnp.pad(a, ((0, 0), (0, D - D0)))
    g = jnp.pad(g, ((0, 0), (0, D - D0)))
    NC, NS = 2, 16
    NW = NC * NS
    npw = N // NW
    GN = 128 // K
    ngroups = npw // GN
    nreg = D0 // 16
    idxr = idx.reshape(N * K // 128, 128)
    mesh = plsc.VectorSubcoreMesh(core_axis_name="c", subcore_axis_name="s")

    @functools.partial(
        pl.kernel, mesh=mesh,
        out_type=jax.ShapeDtypeStruct((N, D), jnp.float32),
        scratch_types=[pltpu.VMEM((ngroups, 128), jnp.int32),
                       pltpu.VMEM((npw, D), jnp.float32),
                       pltpu.VMEM((128, D), jnp.float32),
                       pltpu.VMEM((npw, D), jnp.float32),
                       pltpu.SemaphoreType.DMA],
    )
    def edge_k(a_hbm, g_hbm, idx_hbm, out_hbm, idx_v, a_v, rows_v, out_v, sem):
        wid = lax.axis_index("s") * NC + lax.axis_index("c")
        base = wid * npw
        pltpu.sync_copy(idx_hbm.at[pl.ds(wid * ngroups, ngroups)], idx_v)
        pltpu.sync_copy(a_hbm.at[pl.ds(base, npw)], a_v)
        inv = jnp.float32(1.0 / K)

        def group(gi, _):
            pltpu.async_copy(g_hbm.at[idx_v.at[gi]], rows_v, sem).wait()
            for n in range(GN):
                node = gi * GN + n
                a_regs = [a_v[node, pl.ds(c * 16, 16)] for c in range(nreg)]

                def kbody(kk, acc):
                    r = n * K + kk
                    out = []
                    for c in range(nreg):
                        t = a_regs[c] + rows_v[r, pl.ds(c * 16, 16)]
                        out.append(acc[c] + t / (1.0 + jnp.exp(-t)))
                    return tuple(out)

                acc = lax.fori_loop(
                    0, K, kbody,
                    tuple(jnp.zeros((16,), jnp.float32) for _ in range(nreg)))
                for c in range(nreg):
                    out_v[node, pl.ds(c * 16, 16)] = acc[c] * inv
            return 0

        lax.fori_loop(0, ngroups, group, 0)
        pltpu.sync_copy(out_v, out_hbm.at[pl.ds(base, npw)])

    return edge_k(a, g, idxr)[:, :D0]


def _mid_body(s_ref, w12, b12, daw, dab, dbw, x1_ref, a2_ref, g2_ref):
    x1 = _dot(s_ref[...], w12[...]) + b12[...]
    x1_ref[...] = x1
    a2_ref[...] = _dot(x1, daw[...]) + dab[...]
    g2_ref[...] = _dot(x1, dbw[...])


def _run_mid(s1, p, d2diff, d2b_w):
    row = lambda v: v.reshape(1, -1)
    blk = lambda r, c: pl.BlockSpec((r, c), lambda i: (i, 0))
    return pl.pallas_call(
        _mid_body,
        grid=(8,),
        in_specs=[blk(512, HID), _full((160, 160)), _full((1, 160)),
                  _full((160, 160)), _full((1, 160)), _full((160, 160))],
        out_specs=[blk(512, HID)] * 3,
        out_shape=[jax.ShapeDtypeStruct((N_PFC, HID), jnp.float32)] * 3,
    )(s1, p['d1_2_w'], row(p['d1_2_b']), d2diff, row(p['d2_1_b']), d2b_w)


def _vtx_body(xv_ref, bv_ref, w11, b11, w12, b12, w13, b13,
              w21, b21, w22, b22, euc_ref, t_ref, vf_ref):
    xv = xv_ref[...]
    h = _silu(_dot(xv, w11[...]) + b11[...])
    h = _silu(_dot(h, w12[...]) + b12[...])
    euc = _dot(h, w13[...]) + b13[...]
    euc_ref[...] = euc
    vf = _dot(_silu(_dot(euc, w21[...]) + b21[...]), w22[...]) + b22[...]
    vf_ref[...] = vf
    bv = bv_ref[...]
    for b in range(NB):
        fb = jnp.sum((bv < b).astype(jnp.int32))
        fb = jnp.minimum(fb, N_VTX - 1)
        t_ref[b:b + 1, :] = vf_ref[pl.ds(fb, 1), :]


def _run_vtx(x_vtx, bv, p):
    row = lambda v: v.reshape(1, -1)
    return pl.pallas_call(
        _vtx_body,
        grid=(1,),
        in_specs=[_full((N_VTX, 5)), _full((N_VTX, 1)),
                  _full((5, 40)), _full((1, 40)), _full((40, 80)),
                  _full((1, 80)), _full((80, 160)), _full((1, 160)),
                  _full((160, 320)), _full((1, 320)), _full((320, 160)),
                  _full((1, 160))],
        out_specs=[_full((N_VTX, HID)), _full((NB, HID))],
        out_shape=[jax.ShapeDtypeStruct((N_VTX, HID), jnp.float32),
                   jax.ShapeDtypeStruct((NB, HID), jnp.float32)],
        scratch_shapes=[pltpu.VMEM((N_VTX, HID), jnp.float32)],
    )(x_vtx, bv, p['v1_1_w'], row(p['v1_1_b']), p['v1_2_w'], row(p['v1_2_b']),
      p['v1_3_w'], row(p['v1_3_b']), p['v2_1_w'], row(p['v2_1_b']),
      p['v2_2_w'], row(p['v2_2_b']))


def _final_body(s2_ref, bp_ref, w22d, b22d, p1w, p1b, p2w, p2b, t_ref,
                sc_ref):
    feats2 = _dot(s2_ref[...], w22d[...]) + b22d[...]
    h = _silu(_dot(feats2, p1w[...]) + p1b[...])
    pf = _dot(h, p2w[...]) + p2b[...]
    bp = bp_ref[...]
    oh = (bp == lax.broadcasted_iota(jnp.int32, (1, NB), 1)).astype(
        jnp.float32)
    tgt = _dot(oh, t_ref[...])
    s0 = jnp.sum(pf * tgt, axis=1, keepdims=True)
    sc_ref[...] = jnp.concatenate([s0, -s0], axis=1)


def _run_final(s2, bp, p, t):
    row = lambda v: v.reshape(1, -1)
    blk = lambda r, c: pl.BlockSpec((r, c), lambda i: (i, 0))
    return pl.pallas_call(
        _final_body,
        grid=(8,),
        in_specs=[blk(512, HID), blk(512, 1), _full((160, 160)),
                  _full((1, 160)), _full((160, 320)), _full((1, 320)),
                  _full((320, 160)), _full((1, 160)), _full((NB, HID))],
        out_specs=blk(512, 2),
        out_shape=jax.ShapeDtypeStruct((N_PFC, 2), jnp.float32),
    )(s2, bp, p['d2_2_w'], row(p['d2_2_b']), p['p2_1_w'], row(p['p2_1_b']),
      p['p2_2_w'], row(p['p2_2_b']), t)


def kernel(x_pfc, x_vtx, params, batch_pfc, batch_vtx):
    p = params
    f32 = jnp.float32
    n1w_pad = jnp.concatenate([p['n_1_w'], jnp.zeros((1, 80), f32)], axis=0)
    d1diff = p['d1_1_w'][:HID] - p['d1_1_w'][HID:]
    d2diff = p['d2_1_w'][:HID] - p['d2_1_w'][HID:]
    bp_col = batch_pfc.reshape(-1, 1)
    bp_row = batch_pfc.reshape(1, -1)
    bv_col = batch_vtx.reshape(-1, 1)
    ones = jnp.ones((1, N_PFC), f32)

    x0, a1, g1, chf = _run_prep(x_pfc, p, n1w_pad, d1diff, p['d1_1_w'][HID:])
    idx1 = _run_knn(x0, x0.T, bp_row, bp_col, ones, 32, True)
    s1 = _edge_conv_sc(a1, g1, idx1, 32)
    x1, a2, g2 = _run_mid(s1, p, d2diff, p['d2_1_w'][HID:])
    idx2 = _run_knn(x1, x1.T, bp_row, bp_col, chf.reshape(1, -1), 16, False)
    s2 = _edge_conv_sc(a2, g2, idx2, 16)
    euc, t = _run_vtx(x_vtx, bv_col, p)
    scores = _run_final(s2, bp_col, p, t)
    return scores, x1, euc
